# baseline (speedup 1.0000x reference)
import functools

import jax
import jax.numpy as jnp
from jax import lax
from jax.experimental import pallas as pl
from jax.experimental.pallas import tpu as pltpu
from jax.experimental.pallas import tpu_sc as plsc

D = 52
DP = 64
DE = 10
DEP = 16
K11 = 11
KD = K11 * DP
NSTEP = 3
N = 4000
NT = 2 * N
E1 = 32000
EP = 32768
ET = 2 * EP
NA = 8192
TRASH_U = 8100
TRASH_V = 8150
CH = 128
NCHW = 8
NCH2 = 16
NW = 32
RPS = NA // 16
RB = 1000
RBF = 200

f32 = jnp.float32
i32 = jnp.int32


def _pad2(w, r, c):
    return jnp.zeros((r, c), f32).at[: w.shape[0], : w.shape[1]].set(w)


def _embed_body(x_ref, w_ref, b_ref, o_ref):
    o_ref[...] = jnp.maximum(jnp.dot(x_ref[...], w_ref[0]) + b_ref[0], 0.0)


def _embed(x, w, b, blk):
    rows, c = x.shape
    nb = rows // blk
    per = nb // 2
    return pl.pallas_call(
        _embed_body,
        grid=(nb,),
        in_specs=[
            pl.BlockSpec((blk, c), lambda b_: (b_, 0)),
            pl.BlockSpec((1, c, w.shape[2]), lambda b_: (b_ // per, 0, 0)),
            pl.BlockSpec((1, 1, w.shape[2]), lambda b_: (b_ // per, 0, 0)),
        ],
        out_specs=pl.BlockSpec((blk, w.shape[2]), lambda b_: (b_, 0)),
        out_shape=jax.ShapeDtypeStruct((rows, w.shape[2]), f32),
    )(x, w, b)


def _msg_body(g_ref, ehx_ref, s_ref, w_ref, o_ref):
    g = g_ref[...]
    ge = jnp.concatenate([g] * K11, axis=1)
    z = ge * jnp.dot(ehx_ref[...], s_ref[...])
    o_ref[...] = jnp.dot(z, w_ref[0])


def _msg(g, ehx, smat, wf2):
    blk = 2048
    nb = ET // blk
    per = nb // 2
    return pl.pallas_call(
        _msg_body,
        grid=(nb,),
        in_specs=[
            pl.BlockSpec((blk, DP), lambda b_: (b_, 0)),
            pl.BlockSpec((blk, DEP), lambda b_: (b_, 0)),
            pl.BlockSpec((DEP, KD), lambda b_: (0, 0)),
            pl.BlockSpec((1, KD, DP), lambda b_: (b_ // per, 0, 0)),
        ],
        out_specs=pl.BlockSpec((blk, DP), lambda b_: (b_, 0)),
        out_shape=jax.ShapeDtypeStruct((ET, DP), f32),
    )(g, ehx, smat, wf2)


def _upd_body(a0_ref, a1_ref, h_ref, wm_ref, wh_ref, b_ref, o_ref):
    m = jnp.maximum(a0_ref[0] + a1_ref[0], 0.0)
    o_ref[...] = (
        jnp.dot(m, wm_ref[0]) + jnp.dot(h_ref[...], wh_ref[0]) + b_ref[0]
    )


def _upd(agg2, h, wm2, wh2, bm2):
    nb = NT // RB
    per = nb // 2
    return pl.pallas_call(
        _upd_body,
        grid=(nb,),
        in_specs=[
            pl.BlockSpec((1, RB, DP), lambda b_: (0, b_, 0)),
            pl.BlockSpec((1, RB, DP), lambda b_: (1, b_, 0)),
            pl.BlockSpec((RB, DP), lambda b_: (b_, 0)),
            pl.BlockSpec((1, DP, DP), lambda b_: (b_ // per, 0, 0)),
            pl.BlockSpec((1, DP, DP), lambda b_: (b_ // per, 0, 0)),
            pl.BlockSpec((1, 1, DP), lambda b_: (b_ // per, 0, 0)),
        ],
        out_specs=pl.BlockSpec((RB, DP), lambda b_: (b_, 0)),
        out_shape=jax.ShapeDtypeStruct((NT, DP), f32),
    )(agg2, agg2, h, wm2, wh2, bm2)


def _final_body(hu_ref, xu_ref, hv_ref, xv_ref, bu_ref, bv_ref,
                im_ref, osu_ref, osv_ref, svn_ref, acc_ref):
    i = pl.program_id(0)

    @pl.when(i == 0)
    def _():
        sv = hv_ref[...] + xv_ref[...]
        nv = jnp.sqrt(jnp.sum(sv * sv, axis=1, keepdims=True))
        svn_ref[...] = sv / jnp.maximum(nv, 1e-12)

    su = hu_ref[...] + xu_ref[...]
    nu = jnp.sqrt(jnp.sum(su * su, axis=1, keepdims=True))
    su = su / jnp.maximum(nu, 1e-12)
    svn = svn_ref[...]
    raw = lax.dot_general(su, svn, (((1,), (1,)), ((), ())))
    im = jnp.where(bu_ref[...] == bv_ref[...], raw, 0.0)
    im_ref[...] = im
    osu_ref[:, 0, :] = su
    osu_ref[:, 1, :] = jnp.dot(im, svn)
    contrib = lax.dot_general(im, su, (((0,), (0,)), ((), ())))

    @pl.when(i == 0)
    def _():
        acc_ref[...] = contrib

    @pl.when(i > 0)
    def _():
        acc_ref[...] = acc_ref[...] + contrib

    @pl.when(i == pl.num_programs(0) - 1)
    def _():
        osv_ref[:, 0, :] = svn_ref[...]
        osv_ref[:, 1, :] = acc_ref[...]


def _final(h_all, x_all, bu, bv):
    nb = N // RBF
    return pl.pallas_call(
        _final_body,
        grid=(nb,),
        in_specs=[
            pl.BlockSpec((RBF, DP), lambda b_: (b_, 0)),
            pl.BlockSpec((RBF, DP), lambda b_: (b_, 0)),
            pl.BlockSpec((N, DP), lambda b_: (1, 0)),
            pl.BlockSpec((N, DP), lambda b_: (1, 0)),
            pl.BlockSpec((RBF, 1), lambda b_: (b_, 0)),
            pl.BlockSpec((1, N), lambda b_: (0, 0)),
        ],
        out_specs=[
            pl.BlockSpec((RBF, N), lambda b_: (b_, 0)),
            pl.BlockSpec((RBF, 2, DP), lambda b_: (b_, 0, 0)),
            pl.BlockSpec((N, 2, DP), lambda b_: (0, 0, 0)),
        ],
        out_shape=[
            jax.ShapeDtypeStruct((N, N), f32),
            jax.ShapeDtypeStruct((N, 2, DP), f32),
            jax.ShapeDtypeStruct((N, 2, DP), f32),
        ],
        scratch_shapes=[
            pltpu.VMEM((N, DP), f32),
            pltpu.VMEM((N, DP), f32),
        ],
    )(h_all, x_all, h_all, x_all, bu, bv)


def _sc_mesh():
    return plsc.VectorSubcoreMesh(
        core_axis_name="c", subcore_axis_name="s", num_cores=2, num_subcores=16
    )


def _out_off(wid, j):
    if j < NCHW:
        return (wid * NCHW + j) * CH
    return EP + (wid * NCHW + (j - NCHW)) * CH


def _sc_gather(h_table, src2):
    @functools.partial(
        pl.kernel,
        out_type=jax.ShapeDtypeStruct((ET, DP), f32),
        mesh=_sc_mesh(),
        compiler_params=pltpu.CompilerParams(use_tc_tiling_on_sc=False),
        scratch_types=[
            pltpu.VMEM((NCH2, CH), i32),
            pltpu.VMEM((CH, DP), f32),
            pltpu.VMEM((CH, DP), f32),
            pltpu.VMEM_SHARED((NA, DP), f32),
            pltpu.SemaphoreType.DMA,
            pltpu.SemaphoreType.DMA,
        ],
    )
    def gk(h_hbm, src_hbm, out_hbm, idx_v, rows_a, rows_b, tab_sh,
           sem_a, sem_b):
        c = lax.axis_index("c")
        s = lax.axis_index("s")
        wid = s * 2 + c

        @pl.when(s < 15)
        def _():
            pltpu.sync_copy(h_hbm.at[pl.ds(s * 512, 512)],
                            tab_sh.at[pl.ds(s * 512, 512)])

        @pl.when(s == 15)
        def _():
            pltpu.sync_copy(h_hbm.at[pl.ds(7680, 320)],
                            tab_sh.at[pl.ds(7680, 320)])

        pltpu.sync_copy(src_hbm.at[pl.ds(wid * NCH2, NCH2)], idx_v)
        plsc.subcore_barrier()
        bufs = (rows_a, rows_b)
        sems = (sem_a, sem_b)
        cps = [None, None]
        cps[0] = pltpu.async_copy(tab_sh.at[idx_v.at[0]], bufs[0], sems[0])
        for j in range(NCH2):
            if j + 1 < NCH2:
                cps[(j + 1) % 2] = pltpu.async_copy(
                    tab_sh.at[idx_v.at[j + 1]], bufs[(j + 1) % 2],
                    sems[(j + 1) % 2])
            cps[j % 2].wait()
            pltpu.sync_copy(bufs[j % 2],
                            out_hbm.at[pl.ds(_out_off(wid, j), CH)])

    return gk(h_table, src2)


def _sc_scatter(msg, dst2, zeros_na):
    @functools.partial(
        pl.kernel,
        out_type=jax.ShapeDtypeStruct((2, NA, DP), f32),
        mesh=_sc_mesh(),
        compiler_params=pltpu.CompilerParams(use_tc_tiling_on_sc=False),
        scratch_types=[
            pltpu.VMEM((NCH2, CH), i32),
            pltpu.VMEM((CH, DP), f32),
            pltpu.VMEM((CH, DP), f32),
            pltpu.VMEM_SHARED((NA, DP), f32),
            pltpu.SemaphoreType.DMA,
            pltpu.SemaphoreType.DMA,
        ],
    )
    def sk(msg_hbm, dst_hbm, z_hbm, out_hbm, idx_v, row_a, row_b, agg_sh,
           sem_a, sem_b):
        c = lax.axis_index("c")
        s = lax.axis_index("s")
        wid = s * 2 + c
        pltpu.sync_copy(z_hbm.at[pl.ds(s * RPS, RPS)], agg_sh.at[pl.ds(s * RPS, RPS)])
        pltpu.sync_copy(dst_hbm.at[pl.ds(wid * NCH2, NCH2)], idx_v)
        plsc.subcore_barrier()
        bufs = (row_a, row_b)
        sems = (sem_a, sem_b)
        cps = [None, None]
        cps[0] = pltpu.async_copy(
            msg_hbm.at[pl.ds(_out_off(wid, 0), CH)], bufs[0], sems[0])
        for j in range(NCH2):
            if j + 1 < NCH2:
                cps[(j + 1) % 2] = pltpu.async_copy(
                    msg_hbm.at[pl.ds(_out_off(wid, j + 1), CH)],
                    bufs[(j + 1) % 2], sems[(j + 1) % 2])
            cps[j % 2].wait()
            pltpu.sync_copy(bufs[j % 2], agg_sh.at[idx_v.at[j]], add=True)
        plsc.subcore_barrier()
        pltpu.sync_copy(
            agg_sh.at[pl.ds(s * RPS, RPS)],
            out_hbm.at[c].at[pl.ds(s * RPS, RPS)],
        )

    return sk(msg, dst2, zeros_na)


def kernel(solute_x, solute_edge_index, solute_edge_attr, solvent_x,
           solvent_edge_index, solvent_edge_attr, solute_batch, solvent_batch,
           su_lin0_W, su_lin0_b, su_e1_W, su_e1_b, su_e2_W, su_e2_b,
           su_msg_W, su_msg_b, sv_lin0_W, sv_lin0_b, sv_e1_W, sv_e1_b,
           sv_e2_W, sv_e2_b, sv_msg_W, sv_msg_b):
    def idxhalf(ei, off, trash):
        ei = ei.astype(i32)
        src = (jnp.zeros((EP,), i32).at[:E1].set(ei[0]) + off
               ).reshape(NW, NCHW, CH)
        dst = (jnp.full((EP,), trash, i32).at[:E1].set(ei[1] + off)
               ).reshape(NW, NCHW, CH)
        return src, dst

    src_u, dst_u = idxhalf(solute_edge_index, 0, TRASH_U)
    src_v, dst_v = idxhalf(solvent_edge_index, N, TRASH_V)
    src2 = jnp.concatenate([src_u, src_v], axis=1).reshape(NW * NCH2, CH)
    dst2 = jnp.concatenate([dst_u, dst_v], axis=1).reshape(NW * NCH2, CH)

    x_all = (jnp.zeros((NT, DP), f32)
             .at[:N, :D].set(solute_x).at[N:, :D].set(solvent_x))
    ea_all = (jnp.zeros((ET, DEP), f32)
              .at[:E1, :DE].set(solute_edge_attr)
              .at[EP:EP + E1, :DE].set(solvent_edge_attr))

    def wset(lin0_W, lin0_b, e1_W, e1_b, e2_W, e2_b, msg_W, msg_b):
        w3x = jnp.concatenate(
            [e2_W.reshape(DE, D, D), e2_b.reshape(1, D, D)], axis=0)
        wf = (jnp.zeros((K11, DP, DP), f32).at[:, :D, :D].set(w3x)
              .transpose(0, 2, 1).reshape(KD, DP))
        return dict(
            w0=_pad2(lin0_W, DP, DP), b0=_pad2(lin0_b[None], 1, DP),
            e1=_pad2(e1_W, DEP, DEP),
            b1=_pad2(e1_b[None], 1, DEP).at[0, DE].set(1.0),
            wf=wf,
            wm=_pad2(msg_W[:D], DP, DP), wh=_pad2(msg_W[D:], DP, DP),
            bm=_pad2(msg_b[None], 1, DP),
        )

    wu = wset(su_lin0_W, su_lin0_b, su_e1_W, su_e1_b, su_e2_W, su_e2_b,
              su_msg_W, su_msg_b)
    wv = wset(sv_lin0_W, sv_lin0_b, sv_e1_W, sv_e1_b, sv_e2_W, sv_e2_b,
              sv_msg_W, sv_msg_b)

    def stk(key):
        return jnp.stack([wu[key], wv[key]], axis=0)

    w0s, b0s = stk("w0"), stk("b0")
    e1s, b1s = stk("e1"), stk("b1")
    wfs = stk("wf")
    wms, whs, bms = stk("wm"), stk("wh"), stk("bm")

    smat = jnp.zeros((DEP, KD), f32)
    for _k in range(K11):
        smat = smat.at[_k, _k * DP:(_k + 1) * DP].set(1.0)
    zeros_na = jnp.zeros((NA, DP), f32)
    bu = solute_batch.astype(i32).reshape(N, 1)
    bv = solvent_batch.astype(i32).reshape(1, N)

    h = _embed(x_all, w0s, b0s, RB)
    ehx = _embed(ea_all, e1s, b1s, 4096)
    for _ in range(NSTEP):
        g = _sc_gather(h, src2)
        m = _msg(g, ehx, smat, wfs)
        a = _sc_scatter(m, dst2, zeros_na)
        h = _upd(a, h, wms, whs, bms)

    im, osu, osv = _final(h, x_all, bu, bv)
    out_su = jnp.concatenate([osu[:, 0, :D], osu[:, 1, :D]], axis=1)
    out_sv = jnp.concatenate([osv[:, 0, :D], osv[:, 1, :D]], axis=1)
    return out_su, out_sv, im
